# baseline (device time: 20277 ns/iter reference)
import jax
import jax.numpy as jnp
from jax import lax
from jax.experimental import pallas as pl
from jax.experimental.pallas import tpu as pltpu

_BM = 256


def kernel(x, dy, gamma):
    m, d = x.shape
    nsteps = m // _BM

    def body(x_ref, dy_ref, out_ref, acc_ref, recv_ref, send_sem, recv_sem):
        i = pl.program_id(0)
        my_x = lax.axis_index("x")
        my_y = lax.axis_index("y")
        my_z = lax.axis_index("z")
        peer = (my_x, 1 - my_y, my_z)

        @pl.when(i == 0)
        def _():
            barrier = pltpu.get_barrier_semaphore()
            pl.semaphore_signal(
                barrier, inc=1, device_id=peer,
                device_id_type=pl.DeviceIdType.MESH,
            )
            pl.semaphore_wait(barrier, 1)

        xv = x_ref[:, :]
        dyv = dy_ref[:, :]
        mu = jnp.mean(xv, axis=1, keepdims=True)
        xc = xv - mu
        var = jnp.mean(xc * xc, axis=1, keepdims=True)
        xhat = xc * lax.rsqrt(var + 1e-5)
        partial = jnp.stack(
            [jnp.sum(dyv * xhat, axis=0), jnp.sum(dyv, axis=0)]
        )

        @pl.when(i == 0)
        def _():
            acc_ref[:, :] = partial

        @pl.when(i > 0)
        def _():
            acc_ref[:, :] = acc_ref[:, :] + partial

        @pl.when(i == nsteps - 1)
        def _():
            rdma = pltpu.make_async_remote_copy(
                src_ref=acc_ref,
                dst_ref=recv_ref,
                send_sem=send_sem,
                recv_sem=recv_sem,
                device_id=peer,
                device_id_type=pl.DeviceIdType.MESH,
            )
            rdma.start()
            rdma.wait()
            out_ref[:, :] = acc_ref[:, :] + recv_ref[:, :]

    return pl.pallas_call(
        body,
        grid=(nsteps,),
        out_shape=jax.ShapeDtypeStruct((2, d), jnp.float32),
        in_specs=[
            pl.BlockSpec((_BM, d), lambda i: (i, 0)),
            pl.BlockSpec((_BM, d), lambda i: (i, 0)),
        ],
        out_specs=pl.BlockSpec((2, d), lambda i: (0, 0)),
        scratch_shapes=[
            pltpu.VMEM((2, d), jnp.float32),
            pltpu.VMEM((2, d), jnp.float32),
            pltpu.SemaphoreType.DMA,
            pltpu.SemaphoreType.DMA,
        ],
        compiler_params=pltpu.CompilerParams(collective_id=0),
    )(x, dy)


# device time: 19133 ns/iter; 1.0598x vs baseline; 1.0598x over previous
import jax
import jax.numpy as jnp
from jax import lax
from jax.experimental import pallas as pl
from jax.experimental.pallas import tpu as pltpu

_BM = 1024


def kernel(x, dy, gamma):
    m, d = x.shape
    nsteps = m // _BM

    def body(x_ref, dy_ref, out_ref, acc_ref, recv_ref, send_sem, recv_sem):
        i = pl.program_id(0)
        my_x = lax.axis_index("x")
        my_y = lax.axis_index("y")
        my_z = lax.axis_index("z")
        peer = (my_x, 1 - my_y, my_z)

        @pl.when(i == 0)
        def _():
            barrier = pltpu.get_barrier_semaphore()
            pl.semaphore_signal(
                barrier, inc=1, device_id=peer,
                device_id_type=pl.DeviceIdType.MESH,
            )
            pl.semaphore_wait(barrier, 1)

        xv = x_ref[:, :]
        dyv = dy_ref[:, :]
        mu = jnp.mean(xv, axis=1, keepdims=True)
        xc = xv - mu
        var = jnp.mean(xc * xc, axis=1, keepdims=True)
        xhat = xc * lax.rsqrt(var + 1e-5)
        partial = jnp.stack(
            [jnp.sum(dyv * xhat, axis=0), jnp.sum(dyv, axis=0)]
        )

        @pl.when(i == 0)
        def _():
            acc_ref[:, :] = partial

        @pl.when(i > 0)
        def _():
            acc_ref[:, :] = acc_ref[:, :] + partial

        @pl.when(i == nsteps - 1)
        def _():
            rdma = pltpu.make_async_remote_copy(
                src_ref=acc_ref,
                dst_ref=recv_ref,
                send_sem=send_sem,
                recv_sem=recv_sem,
                device_id=peer,
                device_id_type=pl.DeviceIdType.MESH,
            )
            rdma.start()
            rdma.wait()
            out_ref[:, :] = acc_ref[:, :] + recv_ref[:, :]

    return pl.pallas_call(
        body,
        grid=(nsteps,),
        out_shape=jax.ShapeDtypeStruct((2, d), jnp.float32),
        in_specs=[
            pl.BlockSpec((_BM, d), lambda i: (i, 0)),
            pl.BlockSpec((_BM, d), lambda i: (i, 0)),
        ],
        out_specs=pl.BlockSpec((2, d), lambda i: (0, 0)),
        scratch_shapes=[
            pltpu.VMEM((2, d), jnp.float32),
            pltpu.VMEM((2, d), jnp.float32),
            pltpu.SemaphoreType.DMA,
            pltpu.SemaphoreType.DMA,
        ],
        compiler_params=pltpu.CompilerParams(collective_id=0),
    )(x, dy)


# device time: 17012 ns/iter; 1.1919x vs baseline; 1.1247x over previous
import jax
import jax.numpy as jnp
from jax import lax
from jax.experimental import pallas as pl
from jax.experimental.pallas import tpu as pltpu

N_DEV = 16
_RM = 256


def kernel(x, dy, gamma):
    m, d = x.shape

    def body(
        x_hbm, dy_hbm, out_ref,
        xb, dyb, allp, local_sems, send_sems, recv_sems,
    ):
        my_x = lax.axis_index("x")
        my_y = lax.axis_index("y")
        my_z = lax.axis_index("z")
        my_id = my_x * 8 + my_y * 4 + my_z
        rank = my_x * 4 + my_z
        r0 = rank * _RM

        barrier = pltpu.get_barrier_semaphore()
        for p in range(1, N_DEV):
            t = (my_id + p) % N_DEV
            pl.semaphore_signal(
                barrier, inc=1,
                device_id=(t // 8, (t % 8) // 4, t % 4),
                device_id_type=pl.DeviceIdType.MESH,
            )

        cp_x = pltpu.make_async_copy(
            x_hbm.at[pl.ds(r0, _RM), :], xb, local_sems.at[0]
        )
        cp_dy = pltpu.make_async_copy(
            dy_hbm.at[pl.ds(r0, _RM), :], dyb, local_sems.at[1]
        )
        cp_x.start()
        cp_dy.start()
        cp_x.wait()
        cp_dy.wait()

        xv = xb[:, :]
        dyv = dyb[:, :]
        mu = jnp.mean(xv, axis=1, keepdims=True)
        xc = xv - mu
        var = jnp.mean(xc * xc, axis=1, keepdims=True)
        xhat = xc * lax.rsqrt(var + 1e-5)
        allp[my_id, 0, :] = jnp.sum(dyv * xhat, axis=0)
        allp[my_id, 1, :] = jnp.sum(dyv, axis=0)

        pl.semaphore_wait(barrier, N_DEV - 1)

        sends = []
        for p in range(1, N_DEV):
            t = (my_id + p) % N_DEV
            rdma = pltpu.make_async_remote_copy(
                src_ref=allp.at[my_id],
                dst_ref=allp.at[my_id],
                send_sem=send_sems.at[p],
                recv_sem=recv_sems.at[N_DEV - p],
                device_id=(t // 8, (t % 8) // 4, t % 4),
                device_id_type=pl.DeviceIdType.MESH,
            )
            rdma.start()
            sends.append(rdma)

        for q in range(1, N_DEV):
            j = (my_id + q) % N_DEV
            recv = pltpu.make_async_remote_copy(
                src_ref=allp.at[my_id],
                dst_ref=allp.at[j],
                send_sem=send_sems.at[q],
                recv_sem=recv_sems.at[q],
                device_id=(my_x, my_y, my_z),
                device_id_type=pl.DeviceIdType.MESH,
            )
            recv.wait_recv()

        out_ref[:, :] = jnp.sum(allp[:, :, :], axis=0)

        for rdma in sends:
            rdma.wait_send()

    return pl.pallas_call(
        body,
        out_shape=jax.ShapeDtypeStruct((2, d), jnp.float32),
        in_specs=[
            pl.BlockSpec(memory_space=pl.ANY),
            pl.BlockSpec(memory_space=pl.ANY),
        ],
        out_specs=pl.BlockSpec(memory_space=pltpu.VMEM),
        scratch_shapes=[
            pltpu.VMEM((_RM, d), jnp.float32),
            pltpu.VMEM((_RM, d), jnp.float32),
            pltpu.VMEM((N_DEV, 2, d), jnp.float32),
            pltpu.SemaphoreType.DMA((2,)),
            pltpu.SemaphoreType.DMA((N_DEV,)),
            pltpu.SemaphoreType.DMA((N_DEV,)),
        ],
        compiler_params=pltpu.CompilerParams(collective_id=0),
    )(x, dy)
